# Initial kernel scaffold; baseline (speedup 1.0000x reference)
import jax
import jax.numpy as jnp
from jax.experimental import pallas as pl


def kernel(x, edge_index, edge_weight, W):
    raise NotImplementedError("write your pallas kernel here")


# speedup vs baseline: 6.0167x; 6.0167x over previous
import functools

import jax
import jax.numpy as jnp
from jax import lax
from jax.experimental import pallas as pl
from jax.experimental.pallas import tpu as pltpu
from jax.experimental.pallas import tpu_sc as plsc

N_NODES = 10000
N_EDGES = 320000
DIM = 128

NC = 2
NS = 16
NW = NC * NS
E_PER_W = N_EDGES // NW
CHUNK = 80
N_CHUNKS = E_PER_W // CHUNK
NB = 5
CPB = N_CHUNKS // NB
ACC_CHUNKS = N_NODES // CHUNK


def _sc_aggregate(x, col3, row3, w3):
    mesh = plsc.VectorSubcoreMesh(core_axis_name="c", subcore_axis_name="s")

    @functools.partial(
        pl.kernel,
        out_type=jax.ShapeDtypeStruct((NC, N_NODES, DIM), jnp.float32),
        mesh=mesh,
        scratch_types=[
            pltpu.VMEM_SHARED((N_NODES, DIM), jnp.float32),
            pltpu.VMEM((CPB, CHUNK), jnp.int32),
            pltpu.VMEM((CPB, CHUNK), jnp.int32),
            pltpu.VMEM((CPB, CHUNK), jnp.float32),
            pltpu.VMEM((CHUNK, DIM), jnp.float32),
            pltpu.SemaphoreType.DMA,
        ],
    )
    def agg_kernel(x_hbm, col_hbm, row_hbm, w_hbm, out_hbm,
                   acc, col_b, row_b, w_b, rows_b, sem):
        c = lax.axis_index("c")
        s = lax.axis_index("s")
        wid = s * NC + c

        def zero_row(k, carry2):
            for g in range(DIM // 16):
                rows_b[k, pl.ds(g * 16, 16)] = jnp.zeros((16,), jnp.float32)
            return carry2

        lax.fori_loop(0, CHUNK, zero_row, 0)

        def zero_chunk(i, carry2):
            m = s + i * NS

            @pl.when(m < ACC_CHUNKS)
            def _():
                pltpu.sync_copy(rows_b, acc.at[pl.ds(m * CHUNK, CHUNK)])
            return carry2

        lax.fori_loop(0, (ACC_CHUNKS + NS - 1) // NS, zero_chunk, 0)
        plsc.subcore_barrier()

        def batch_body(b, carry):
            pltpu.sync_copy(col_hbm.at[wid, b], col_b)
            pltpu.sync_copy(row_hbm.at[wid, b], row_b)
            pltpu.sync_copy(w_hbm.at[wid, b], w_b)

            def chunk_body(j, carry1):
                pltpu.async_copy(x_hbm.at[col_b.at[j]], rows_b, sem).wait()

                def grp_body(g, carry2):
                    wv = w_b[j, pl.ds(g * 16, 16)]
                    for l in range(16):
                        k = g * 16 + l
                        w = wv[l]
                        for gg in range(DIM // 16):
                            sl = pl.ds(gg * 16, 16)
                            rows_b[k, sl] = rows_b[k, sl] * w
                    return carry2

                lax.fori_loop(0, CHUNK // 16, grp_body, 0)
                pltpu.sync_copy(rows_b, acc.at[row_b.at[j]], add=True)
                return carry1

            lax.fori_loop(0, CPB, chunk_body, 0)
            return carry

        lax.fori_loop(0, NB, batch_body, 0)
        plsc.subcore_barrier()

        def pub_chunk(i, carry2):
            m = s + i * NS

            @pl.when(m < ACC_CHUNKS)
            def _():
                pltpu.sync_copy(acc.at[pl.ds(m * CHUNK, CHUNK)], rows_b)
                pltpu.sync_copy(rows_b, out_hbm.at[c, pl.ds(m * CHUNK, CHUNK)])
            return carry2

        lax.fori_loop(0, (ACC_CHUNKS + NS - 1) // NS, pub_chunk, 0)

    return agg_kernel(x, col3, row3, w3)


def _tc_combine_matmul(partials, W):
    BLK = 1000

    def mm_body(p_ref, w_ref, o_ref):
        a = p_ref[0] + p_ref[1]
        o_ref[...] = lax.dot_general(
            a, w_ref[...], (((1,), (1,)), ((), ())),
            preferred_element_type=jnp.float32,
            precision=lax.Precision.HIGHEST)

    return pl.pallas_call(
        mm_body,
        grid=(N_NODES // BLK,),
        in_specs=[
            pl.BlockSpec((NC, BLK, DIM), lambda i: (0, i, 0)),
            pl.BlockSpec((DIM, DIM), lambda i: (0, 0)),
        ],
        out_specs=pl.BlockSpec((BLK, DIM), lambda i: (i, 0)),
        out_shape=jax.ShapeDtypeStruct((N_NODES, DIM), jnp.float32),
    )(partials, W)


def kernel(x, edge_index, edge_weight, W):
    col3 = edge_index[1].astype(jnp.int32).reshape(NW, NB, CPB, CHUNK)
    row3 = edge_index[0].astype(jnp.int32).reshape(NW, NB, CPB, CHUNK)
    w3 = edge_weight.reshape(NW, NB, CPB, CHUNK)
    partials = _sc_aggregate(x, col3, row3, w3)
    return _tc_combine_matmul(partials, W)
